# baseline (speedup 1.0000x reference)
import functools

import jax
import jax.numpy as jnp
from jax.experimental import pallas as pl
from jax.experimental.pallas import tpu as pltpu

_KS = 2
_BN_EPS = 1e-5


def _cast_split_kernel(x_ref, o_ref, *, w_len):
    v = x_ref[...].astype(jnp.bfloat16)
    o_ref[:, 0, :, :] = v[:, :, :w_len]
    o_ref[:, 1, :, :] = v[:, :, w_len:]


def _conv_stats_kernel(x_ref, w_ref, b_ref, y_ref, sum_ref, ssq_ref, *,
                       cin, how):
    v = x_ref[...]
    slices = []
    for quarter in range(4):
        vi = v[:, quarter * how:(quarter + 1) * how]
        lo = jax.lax.bitcast_convert_type(vi.astype(jnp.int16),
                                          jnp.bfloat16)
        hi = jax.lax.bitcast_convert_type(
            jax.lax.shift_right_logical(vi, jnp.int32(16)).astype(jnp.int16),
            jnp.bfloat16)
        slices.append(lo)
        slices.append(hi)
    p = jnp.concatenate(slices, axis=0)

    y = jnp.dot(w_ref[...], p, preferred_element_type=jnp.float32)
    y = y + b_ref[...]
    y_ref[...] = y
    sum_ref[...] = jnp.sum(y, axis=1, keepdims=True)
    ssq_ref[...] = jnp.sum(y * y, axis=1, keepdims=True)


def _bn_prelu_kernel(y_ref, scale_ref, shift_ref, alpha_ref, o_ref):
    z = y_ref[...] * scale_ref[...] + shift_ref[...]
    o_ref[...] = jnp.where(z > 0, z, alpha_ref[...] * z)


def kernel(x, conv_w, conv_b, bn_gamma, bn_beta, prelu_alpha):
    N, Cin, D, H, W = x.shape
    Cout = conv_w.shape[0]
    Do, Ho, Wo = D // _KS, H // _KS, W // _KS
    spatial = Do * Ho * Wo
    how = Ho * Wo

    x_v = x.reshape(N, Cin, D, Ho, _KS * W)
    xb = pl.pallas_call(
        functools.partial(_cast_split_kernel, w_len=W),
        out_shape=jax.ShapeDtypeStruct((N, Cin, D, _KS, Ho, W),
                                       jnp.bfloat16),
        grid=(N, D),
        in_specs=[
            pl.BlockSpec((None, Cin, None, Ho, _KS * W),
                         lambda n, d: (n, 0, d, 0, 0)),
        ],
        out_specs=pl.BlockSpec((None, Cin, None, _KS, Ho, W),
                               lambda n, d: (n, 0, d, 0, 0, 0)),
        compiler_params=pltpu.CompilerParams(
            dimension_semantics=("parallel", "parallel")),
    )(x_v)

    xi = jax.lax.bitcast_convert_type(
        xb.reshape(N, Cin, D, _KS, Ho, Wo, _KS), jnp.int32)
    xi = xi.reshape(N, Cin, D * _KS * how)

    w_r = conv_w.transpose(2, 3, 4, 1, 0).reshape(8 * Cin, Cout)
    w_r = w_r.T.astype(jnp.bfloat16)
    b_col = conv_b.reshape(Cout, 1)

    y_t, psum, pssq = pl.pallas_call(
        functools.partial(_conv_stats_kernel, cin=Cin, how=how),
        out_shape=(
            jax.ShapeDtypeStruct((N, Cout, spatial), jnp.float32),
            jax.ShapeDtypeStruct((N * Do, Cout, 1), jnp.float32),
            jax.ShapeDtypeStruct((N * Do, Cout, 1), jnp.float32),
        ),
        grid=(N, Do),
        in_specs=[
            pl.BlockSpec((None, Cin, 4 * how), lambda n, d: (n, 0, d)),
            pl.BlockSpec((Cout, 8 * Cin), lambda n, d: (0, 0)),
            pl.BlockSpec((Cout, 1), lambda n, d: (0, 0)),
        ],
        out_specs=(
            pl.BlockSpec((None, Cout, how), lambda n, d: (n, 0, d)),
            pl.BlockSpec((None, Cout, 1),
                         lambda n, d, do=Do: (n * do + d, 0, 0)),
            pl.BlockSpec((None, Cout, 1),
                         lambda n, d, do=Do: (n * do + d, 0, 0)),
        ),
        compiler_params=pltpu.CompilerParams(
            dimension_semantics=("parallel", "parallel")),
    )(xi, w_r, b_col)

    cnt = jnp.float32(N * spatial)
    s = jnp.sum(psum, axis=(0, 2))
    sq = jnp.sum(pssq, axis=(0, 2))
    mean = s / cnt
    var = jnp.maximum(sq / cnt - mean * mean, 0.0)
    inv = jax.lax.rsqrt(var + _BN_EPS)
    scale = (bn_gamma * inv).reshape(Cout, 1)
    shift = (bn_beta - mean * bn_gamma * inv).reshape(Cout, 1)

    tile_s = min(spatial, 4096)
    grid_s = spatial // tile_s
    out_t = pl.pallas_call(
        _bn_prelu_kernel,
        out_shape=jax.ShapeDtypeStruct((N, Cout, spatial), jnp.float32),
        grid=(N, grid_s),
        in_specs=[
            pl.BlockSpec((None, Cout, tile_s), lambda n, s: (n, 0, s)),
            pl.BlockSpec((Cout, 1), lambda n, s: (0, 0)),
            pl.BlockSpec((Cout, 1), lambda n, s: (0, 0)),
            pl.BlockSpec((1, 1), lambda n, s: (0, 0)),
        ],
        out_specs=pl.BlockSpec((None, Cout, tile_s), lambda n, s: (n, 0, s)),
        compiler_params=pltpu.CompilerParams(
            dimension_semantics=("parallel", "parallel")),
    )(y_t, scale, shift, prelu_alpha)

    return out_t.reshape(N, Cout, Do, Ho, Wo)


# speedup vs baseline: 2.0762x; 2.0762x over previous
import functools

import jax
import jax.numpy as jnp
from jax.experimental import pallas as pl
from jax.experimental.pallas import tpu as pltpu

_KS = 2
_BN_EPS = 1e-5


def _lo16(v):
    return jax.lax.bitcast_convert_type(v.astype(jnp.int16), jnp.bfloat16)


def _hi16(v):
    return jax.lax.bitcast_convert_type(
        jax.lax.shift_right_logical(v, jnp.int32(16)).astype(jnp.int16),
        jnp.bfloat16)


def _im2col_kernel(x_ref, o_ref):
    v = x_ref[...].astype(jnp.bfloat16)
    vt = jnp.swapaxes(v, 1, 2)
    vi = pltpu.bitcast(vt, jnp.int32)
    for kw in range(_KS):
        b = _lo16(vi) if kw == 0 else _hi16(vi)
        bt = jnp.swapaxes(b, 1, 2)
        bi = pltpu.bitcast(bt, jnp.int32)
        o_ref[:, kw, 0, :, :] = _lo16(bi)
        o_ref[:, kw, 1, :, :] = _hi16(bi)


def _conv_stats_kernel(x_ref, w_ref, b_ref, y_ref, sum_ref, ssq_ref, *,
                       how):
    v = x_ref[...]
    p = jnp.concatenate(
        [v[:, q * how:(q + 1) * how] for q in range(8)], axis=0)
    y = jnp.dot(w_ref[...], p, preferred_element_type=jnp.float32)
    y = y + b_ref[...]
    y_ref[...] = y
    sum_ref[...] = jnp.sum(y, axis=1, keepdims=True)
    ssq_ref[...] = jnp.sum(y * y, axis=1, keepdims=True)


def _bn_prelu_kernel(y_ref, scale_ref, shift_ref, alpha_ref, o_ref):
    z = y_ref[...] * scale_ref[...] + shift_ref[...]
    o_ref[...] = jnp.where(z > 0, z, alpha_ref[...] * z)


def kernel(x, conv_w, conv_b, bn_gamma, bn_beta, prelu_alpha):
    N, Cin, D, H, W = x.shape
    Cout = conv_w.shape[0]
    Do, Ho, Wo = D // _KS, H // _KS, W // _KS
    spatial = Do * Ho * Wo
    how = Ho * Wo

    xp = pl.pallas_call(
        _im2col_kernel,
        out_shape=jax.ShapeDtypeStruct((N, Cin, D, _KS, _KS, Ho, Wo),
                                       jnp.bfloat16),
        grid=(N, D),
        in_specs=[
            pl.BlockSpec((None, Cin, None, H, W),
                         lambda n, d: (n, 0, d, 0, 0)),
        ],
        out_specs=pl.BlockSpec((None, Cin, None, _KS, _KS, Ho, Wo),
                               lambda n, d: (n, 0, d, 0, 0, 0, 0)),
        compiler_params=pltpu.CompilerParams(
            dimension_semantics=("parallel", "parallel")),
    )(x)

    xp = xp.reshape(N, Cin, D * 4 * how)

    w_r = conv_w.transpose(2, 4, 3, 1, 0).reshape(8 * Cin, Cout)
    w_r = w_r.T.astype(jnp.bfloat16)
    b_col = conv_b.reshape(Cout, 1)

    y_t, psum, pssq = pl.pallas_call(
        functools.partial(_conv_stats_kernel, how=how),
        out_shape=(
            jax.ShapeDtypeStruct((N, Cout, spatial), jnp.float32),
            jax.ShapeDtypeStruct((N * Do, Cout, 1), jnp.float32),
            jax.ShapeDtypeStruct((N * Do, Cout, 1), jnp.float32),
        ),
        grid=(N, Do),
        in_specs=[
            pl.BlockSpec((None, Cin, 8 * how), lambda n, d: (n, 0, d)),
            pl.BlockSpec((Cout, 8 * Cin), lambda n, d: (0, 0)),
            pl.BlockSpec((Cout, 1), lambda n, d: (0, 0)),
        ],
        out_specs=(
            pl.BlockSpec((None, Cout, how), lambda n, d: (n, 0, d)),
            pl.BlockSpec((None, Cout, 1),
                         lambda n, d, do=Do: (n * do + d, 0, 0)),
            pl.BlockSpec((None, Cout, 1),
                         lambda n, d, do=Do: (n * do + d, 0, 0)),
        ),
        compiler_params=pltpu.CompilerParams(
            dimension_semantics=("parallel", "parallel")),
    )(xp, w_r, b_col)

    cnt = jnp.float32(N * spatial)
    s = jnp.sum(psum, axis=(0, 2))
    sq = jnp.sum(pssq, axis=(0, 2))
    mean = s / cnt
    var = jnp.maximum(sq / cnt - mean * mean, 0.0)
    inv = jax.lax.rsqrt(var + _BN_EPS)
    scale = (bn_gamma * inv).reshape(Cout, 1)
    shift = (bn_beta - mean * bn_gamma * inv).reshape(Cout, 1)

    tile_s = min(spatial, 4096)
    grid_s = spatial // tile_s
    out_t = pl.pallas_call(
        _bn_prelu_kernel,
        out_shape=jax.ShapeDtypeStruct((N, Cout, spatial), jnp.float32),
        grid=(N, grid_s),
        in_specs=[
            pl.BlockSpec((None, Cout, tile_s), lambda n, s: (n, 0, s)),
            pl.BlockSpec((Cout, 1), lambda n, s: (0, 0)),
            pl.BlockSpec((Cout, 1), lambda n, s: (0, 0)),
            pl.BlockSpec((1, 1), lambda n, s: (0, 0)),
        ],
        out_specs=pl.BlockSpec((None, Cout, tile_s), lambda n, s: (n, 0, s)),
        compiler_params=pltpu.CompilerParams(
            dimension_semantics=("parallel", "parallel")),
    )(y_t, scale, shift, prelu_alpha)

    return out_t.reshape(N, Cout, Do, Ho, Wo)


# speedup vs baseline: 2.6850x; 1.2932x over previous
import functools

import jax
import jax.numpy as jnp
from jax.experimental import pallas as pl
from jax.experimental.pallas import tpu as pltpu

_KS = 2
_BN_EPS = 1e-5


def _lo16(v):
    return jax.lax.bitcast_convert_type(v.astype(jnp.int16), jnp.bfloat16)


def _hi16(v):
    return jax.lax.bitcast_convert_type(
        jax.lax.shift_right_logical(v, jnp.int32(16)).astype(jnp.int16),
        jnp.bfloat16)


def _im2col_kernel(x_ref, o_ref):
    v = x_ref[...].astype(jnp.bfloat16)
    vt = jnp.swapaxes(v, 2, 3)
    vi = pltpu.bitcast(vt, jnp.int32)
    for kw in range(_KS):
        b = _lo16(vi) if kw == 0 else _hi16(vi)
        bt = jnp.swapaxes(b, 2, 3)
        bi = pltpu.bitcast(bt, jnp.int32)
        o_ref[:, :, kw, 0, :, :] = _lo16(bi)
        o_ref[:, :, kw, 1, :, :] = _hi16(bi)


def _conv_stats_kernel(x_ref, w_ref, b_ref, y_ref, sum_ref, ssq_ref, *,
                       how, dd):
    v = x_ref[...]
    p = jnp.concatenate(
        [jnp.concatenate(
            [v[:, (d * 8 + q) * how:(d * 8 + q + 1) * how]
             for d in range(dd)], axis=1)
         for q in range(8)], axis=0)
    y = jnp.dot(w_ref[...], p, preferred_element_type=jnp.float32)
    y = y + b_ref[...]
    y_ref[...] = y.astype(jnp.bfloat16)
    sum_ref[...] = jnp.sum(y, axis=1, keepdims=True)
    ssq_ref[...] = jnp.sum(y * y, axis=1, keepdims=True)


def _bn_prelu_kernel(y_ref, scale_ref, shift_ref, alpha_ref, o_ref):
    z = y_ref[...].astype(jnp.float32) * scale_ref[...] + shift_ref[...]
    o_ref[...] = jnp.where(z > 0, z, alpha_ref[...] * z)


def kernel(x, conv_w, conv_b, bn_gamma, bn_beta, prelu_alpha):
    N, Cin, D, H, W = x.shape
    Cout = conv_w.shape[0]
    Do, Ho, Wo = D // _KS, H // _KS, W // _KS
    spatial = Do * Ho * Wo
    how = Ho * Wo

    xp = pl.pallas_call(
        _im2col_kernel,
        out_shape=jax.ShapeDtypeStruct((N, Cin, Do, _KS, _KS, _KS, Ho, Wo),
                                       jnp.bfloat16),
        grid=(N, Do),
        in_specs=[
            pl.BlockSpec((None, Cin, _KS, H, W),
                         lambda n, d: (n, 0, d, 0, 0)),
        ],
        out_specs=pl.BlockSpec(
            (None, Cin, None, _KS, _KS, _KS, Ho, Wo),
            lambda n, d: (n, 0, d, 0, 0, 0, 0, 0)),
        compiler_params=pltpu.CompilerParams(
            dimension_semantics=("parallel", "parallel")),
    )(x.reshape(N, Cin, D, H, W))

    xp = xp.reshape(N, Cin, Do * 8 * how)

    w_r = conv_w.transpose(2, 4, 3, 1, 0).reshape(8 * Cin, Cout)
    w_r = w_r.T.astype(jnp.bfloat16)
    b_col = conv_b.reshape(Cout, 1)

    dd = 4 if Do % 4 == 0 else 1
    grid_d = Do // dd
    y_t, psum, pssq = pl.pallas_call(
        functools.partial(_conv_stats_kernel, how=how, dd=dd),
        out_shape=(
            jax.ShapeDtypeStruct((N, Cout, spatial), jnp.bfloat16),
            jax.ShapeDtypeStruct((N * grid_d, Cout, 1), jnp.float32),
            jax.ShapeDtypeStruct((N * grid_d, Cout, 1), jnp.float32),
        ),
        grid=(N, grid_d),
        in_specs=[
            pl.BlockSpec((None, Cin, dd * 8 * how), lambda n, d: (n, 0, d)),
            pl.BlockSpec((Cout, 8 * Cin), lambda n, d: (0, 0)),
            pl.BlockSpec((Cout, 1), lambda n, d: (0, 0)),
        ],
        out_specs=(
            pl.BlockSpec((None, Cout, dd * how), lambda n, d: (n, 0, d)),
            pl.BlockSpec((None, Cout, 1),
                         lambda n, d, gd=grid_d: (n * gd + d, 0, 0)),
            pl.BlockSpec((None, Cout, 1),
                         lambda n, d, gd=grid_d: (n * gd + d, 0, 0)),
        ),
        compiler_params=pltpu.CompilerParams(
            dimension_semantics=("parallel", "parallel")),
    )(xp, w_r, b_col)

    cnt = jnp.float32(N * spatial)
    s = jnp.sum(psum, axis=(0, 2))
    sq = jnp.sum(pssq, axis=(0, 2))
    mean = s / cnt
    var = jnp.maximum(sq / cnt - mean * mean, 0.0)
    inv = jax.lax.rsqrt(var + _BN_EPS)
    scale = (bn_gamma * inv).reshape(Cout, 1)
    shift = (bn_beta - mean * bn_gamma * inv).reshape(Cout, 1)

    tile_s = min(spatial, 8192)
    grid_s = spatial // tile_s
    out_t = pl.pallas_call(
        _bn_prelu_kernel,
        out_shape=jax.ShapeDtypeStruct((N, Cout, spatial), jnp.float32),
        grid=(N, grid_s),
        in_specs=[
            pl.BlockSpec((None, Cout, tile_s), lambda n, s: (n, 0, s)),
            pl.BlockSpec((Cout, 1), lambda n, s: (0, 0)),
            pl.BlockSpec((Cout, 1), lambda n, s: (0, 0)),
            pl.BlockSpec((1, 1), lambda n, s: (0, 0)),
        ],
        out_specs=pl.BlockSpec((None, Cout, tile_s), lambda n, s: (n, 0, s)),
        compiler_params=pltpu.CompilerParams(
            dimension_semantics=("parallel", "parallel")),
    )(y_t, scale, shift, prelu_alpha)

    return out_t.reshape(N, Cout, Do, Ho, Wo)
